# Initial kernel scaffold; baseline (speedup 1.0000x reference)
import jax
import jax.numpy as jnp
from jax.experimental import pallas as pl


def kernel(x, lora_indices, weight, lora_a_stacked_2d, lora_b_stacked):
    raise NotImplementedError("write your pallas kernel here")


# speedup vs baseline: 1.8581x; 1.8581x over previous
import jax
import jax.numpy as jnp
from jax import lax
from jax.experimental import pallas as pl
from jax.experimental.pallas import tpu as pltpu
from jax.experimental.pallas import tpu_sc as plsc

ORG_VOCAB = 100000
EXTRA_VOCAB = 256
FULL_VOCAB = ORG_VOCAB + EXTRA_VOCAB
EMBED_DIM = 1024
MAX_L = 8
RANK = 16
T = 16384

NC = 2
NS = 16
NW = NC * NS
B_PER_W = T // NW
GCHUNK = 128

TOK_BLK = 256
NBLK = T // TOK_BLK


def _sc_body(x_hbm, l_hbm, lora_a_hbm, arows_hbm, bidx_hbm,
             x_v, l_v, idx_v, bidx_v, arows_v, sem):
    wid = lax.axis_index("s") * NC + lax.axis_index("c")
    base = wid * B_PER_W
    pltpu.sync_copy(x_hbm.at[pl.ds(base, B_PER_W)], x_v)
    pltpu.sync_copy(l_hbm.at[pl.ds(base, B_PER_W)], l_v)

    def step(i, carry):
        xs = x_v[pl.ds(i * 16, 16)]
        ls = l_v[pl.ds(i * 16, 16)]
        idx_v[pl.ds(i * 16, 16)] = xs + ls * FULL_VOCAB
        extra = jnp.where(xs > ORG_VOCAB - 1, ls * EXTRA_VOCAB, 0)
        bidx_v[pl.ds(i * 16, 16)] = xs + extra
        return carry

    lax.fori_loop(0, B_PER_W // 16, step, 0)

    copies = []
    for j in range(B_PER_W // GCHUNK):
        copies.append(pltpu.async_copy(
            lora_a_hbm.at[idx_v.at[pl.ds(j * GCHUNK, GCHUNK)]],
            arows_v.at[pl.ds(j * GCHUNK, GCHUNK)],
            sem,
        ))
    for cp in copies:
        cp.wait()

    pltpu.sync_copy(arows_v, arows_hbm.at[pl.ds(base, B_PER_W)])
    pltpu.sync_copy(bidx_v, bidx_hbm.at[pl.ds(base, B_PER_W)])


def _sc_gather(x, lora_indices, lora_a_stacked_2d):
    mesh = plsc.VectorSubcoreMesh(core_axis_name="c", subcore_axis_name="s")
    return pl.kernel(
        _sc_body,
        out_type=[
            jax.ShapeDtypeStruct((T, RANK), jnp.float32),
            jax.ShapeDtypeStruct((T,), jnp.int32),
        ],
        mesh=mesh,
        scratch_types=[
            pltpu.VMEM((B_PER_W,), jnp.int32),
            pltpu.VMEM((B_PER_W,), jnp.int32),
            pltpu.VMEM((B_PER_W,), jnp.int32),
            pltpu.VMEM((B_PER_W,), jnp.int32),
            pltpu.VMEM((B_PER_W, RANK), jnp.float32),
            pltpu.SemaphoreType.DMA,
        ],
        compiler_params=pltpu.CompilerParams(use_tc_tiling_on_sc=False),
    )(x, lora_indices, lora_a_stacked_2d)


def _tc_body(bidx_sm, a_ref, l_ref, bflat_ref, weight_hbm, out_ref,
             base_vmem, sem):
    i = pl.program_id(0)
    tok0 = i * TOK_BLK

    def issue(j, carry):
        row = bidx_sm[tok0 + j]
        pltpu.make_async_copy(
            weight_hbm.at[pl.ds(row, 1)],
            base_vmem.at[pl.ds(j, 1)],
            sem,
        ).start()
        return carry

    lax.fori_loop(0, TOK_BLK, issue, 0, unroll=8)

    a = a_ref[...]
    at8 = jnp.concatenate([a] * MAX_L, axis=1)
    lv = l_ref[...]
    col = lax.broadcasted_iota(jnp.int32, (TOK_BLK, MAX_L * RANK), 1) // RANK
    a_scat = jnp.where(col == lv, at8, 0.0)
    lora = jnp.dot(a_scat, bflat_ref[...], preferred_element_type=jnp.float32)

    pltpu.make_async_copy(
        weight_hbm.at[pl.ds(0, TOK_BLK)], base_vmem, sem,
    ).wait()
    out_ref[...] = base_vmem[...] + lora


def _tc_call(bidx, a_rows, l2d, bflat, weight):
    grid_spec = pltpu.PrefetchScalarGridSpec(
        num_scalar_prefetch=1,
        grid=(NBLK,),
        in_specs=[
            pl.BlockSpec((TOK_BLK, RANK), lambda i, b: (i, 0)),
            pl.BlockSpec((TOK_BLK, 1), lambda i, b: (i, 0)),
            pl.BlockSpec((MAX_L * RANK, EMBED_DIM), lambda i, b: (0, 0)),
            pl.BlockSpec(memory_space=pltpu.MemorySpace.HBM),
        ],
        out_specs=pl.BlockSpec((TOK_BLK, EMBED_DIM), lambda i, b: (i, 0)),
        scratch_shapes=[
            pltpu.VMEM((TOK_BLK, EMBED_DIM), jnp.float32),
            pltpu.SemaphoreType.DMA,
        ],
    )
    return pl.pallas_call(
        _tc_body,
        grid_spec=grid_spec,
        out_shape=jax.ShapeDtypeStruct((T, EMBED_DIM), jnp.float32),
    )(bidx, a_rows, l2d, bflat, weight)


def kernel(x, lora_indices, weight, lora_a_stacked_2d, lora_b_stacked):
    a_rows, bidx = _sc_gather(x, lora_indices, lora_a_stacked_2d)
    bflat = jnp.transpose(lora_b_stacked[:, 0], (0, 2, 1)).reshape(
        MAX_L * RANK, EMBED_DIM)
    l2d = lora_indices.reshape(T, 1)
    return _tc_call(bidx, a_rows, l2d, bflat, weight)
